# Initial kernel scaffold; baseline (speedup 1.0000x reference)
import jax
import jax.numpy as jnp
from jax.experimental import pallas as pl


def kernel(inputs):
    raise NotImplementedError("write your pallas kernel here")


# speedup vs baseline: 54.7286x; 54.7286x over previous
import functools

import jax
import jax.numpy as jnp
from jax import lax
from jax.experimental import pallas as pl
from jax.experimental.pallas import tpu as pltpu
from jax.experimental.pallas import tpu_sc as plsc

ROWS = 4096
COLS = 8192
HALF = COLS // 2
NW = 32
CW = HALF // NW
NG = CW // 16
CHUNK = 256
NCHUNK = ROWS // CHUNK


def _body(in_hbm, out_hbm, buf_a, buf_b, out_v):
    core = lax.axis_index("c")
    sub = lax.axis_index("s")
    wid = core * 16 + sub
    col0 = wid * CW

    neg = jnp.full((16,), -jnp.inf, jnp.float32)
    zero_i = jnp.zeros((16,), jnp.int32)

    def scan_top2(col_base, buf):
        def chunk_body(k, carry):
            pltpu.sync_copy(
                in_hbm.at[pl.ds(k * CHUNK, CHUNK), pl.ds(col_base, CW)], buf)

            def row_body(r, c):
                v0s, v1s, i0s = c
                rv = jnp.full((16,), k * CHUNK + r, jnp.int32)
                nv0, nv1, ni0 = [], [], []
                for g in range(NG):
                    x = buf[r, pl.ds(g * 16, 16)]
                    v0, v1, i0 = v0s[g], v1s[g], i0s[g]
                    gt = x > v0
                    nv1.append(jnp.maximum(v1, jnp.minimum(x, v0)))
                    ni0.append(jnp.where(gt, rv, i0))
                    nv0.append(jnp.maximum(v0, x))
                return (tuple(nv0), tuple(nv1), tuple(ni0))

            return lax.fori_loop(0, CHUNK, row_body, carry)

        init = (tuple(neg for _ in range(NG)),
                tuple(neg for _ in range(NG)),
                tuple(zero_i for _ in range(NG)))
        return lax.fori_loop(0, NCHUNK, chunk_body, init)

    def scan_argmax(col_base, buf):
        def chunk_body(k, carry):
            pltpu.sync_copy(
                in_hbm.at[pl.ds(k * CHUNK, CHUNK), pl.ds(col_base, CW)], buf)

            def row_body(r, c):
                ms, i1s = c
                rv = jnp.full((16,), k * CHUNK + r, jnp.int32)
                nm, ni1 = [], []
                for g in range(NG):
                    x = buf[r, pl.ds(g * 16, 16)]
                    m, i1 = ms[g], i1s[g]
                    gt = x > m
                    ni1.append(jnp.where(gt, rv, i1))
                    nm.append(jnp.maximum(m, x))
                return (tuple(nm), tuple(ni1))

            return lax.fori_loop(0, CHUNK, row_body, carry)

        init = (tuple(neg for _ in range(NG)),
                tuple(zero_i for _ in range(NG)))
        return lax.fori_loop(0, NCHUNK, chunk_body, init)

    v0s, v1s, i0s = scan_top2(col0, buf_a)
    _, i1s = scan_argmax(HALF + col0, buf_b)

    pen_v = jnp.full((16,), -1e16, jnp.float32)
    zero_f = jnp.zeros((16,), jnp.float32)
    for g in range(NG):
        v0, v1 = v0s[g], v1s[g]
        pen = jnp.where(i0s[g] == i1s[g], pen_v, zero_f)
        out_v[pl.ds(g * 16, 16)] = jnp.maximum(v0 + v0 + pen, v0 + v1)
    pltpu.sync_copy(out_v, out_hbm.at[pl.ds(col0, CW)])


@jax.jit
def _run(inputs):
    mesh = plsc.VectorSubcoreMesh(core_axis_name="c", subcore_axis_name="s")
    f = pl.kernel(
        _body,
        out_type=jax.ShapeDtypeStruct((HALF,), jnp.float32),
        mesh=mesh,
        scratch_types=[
            pltpu.VMEM((CHUNK, CW), jnp.float32),
            pltpu.VMEM((CHUNK, CW), jnp.float32),
            pltpu.VMEM((CW,), jnp.float32),
        ],
    )
    return f(inputs)


def kernel(inputs):
    return _run(inputs).reshape(1, HALF)


# speedup vs baseline: 82.9400x; 1.5155x over previous
import functools

import jax
import jax.numpy as jnp
from jax import lax
from jax.experimental import pallas as pl
from jax.experimental.pallas import tpu as pltpu
from jax.experimental.pallas import tpu_sc as plsc

ROWS = 4096
COLS = 8192
HALF = COLS // 2
NW = 32
CW = HALF // NW
NG = CW // 16
CHUNK = 256
NCHUNK = ROWS // CHUNK


def _chunk_copy(in_hbm, col_base, k, buf, sem):
    return pltpu.make_async_copy(
        in_hbm.at[pl.ds(k * CHUNK, CHUNK), pl.ds(col_base, CW)], buf, sem)


def _scan(in_hbm, col_base, bufs, sems, init, row_body):
    _chunk_copy(in_hbm, col_base, 0, bufs[0], sems[0]).start()

    def outer(t, carry):
        for b in range(2):
            k = t * 2 + b
            _chunk_copy(in_hbm, col_base, k, bufs[b], sems[b]).wait()

            @pl.when(k + 1 < NCHUNK)
            def _():
                _chunk_copy(in_hbm, col_base, k + 1,
                            bufs[1 - b], sems[1 - b]).start()

            carry = lax.fori_loop(
                0, CHUNK, functools.partial(row_body, bufs[b], k), carry)
        return carry

    return lax.fori_loop(0, NCHUNK // 2, outer, init)


def _body(in_hbm, out_hbm, buf_a, buf_b, out_v, sem_a, sem_b):
    core = lax.axis_index("c")
    sub = lax.axis_index("s")
    wid = core * 16 + sub
    col0 = wid * CW

    bufs = (buf_a, buf_b)
    sems = (sem_a, sem_b)

    neg = jnp.full((16,), -jnp.inf, jnp.float32)
    zero_i = jnp.zeros((16,), jnp.int32)

    def top2_row(buf, k, r, c):
        v0s, v1s, i0s = c
        rv = jnp.full((16,), k * CHUNK + r, jnp.int32)
        nv0, nv1, ni0 = [], [], []
        for g in range(NG):
            x = buf[r, pl.ds(g * 16, 16)]
            v0, v1, i0 = v0s[g], v1s[g], i0s[g]
            gt = x > v0
            nv1.append(jnp.maximum(v1, jnp.minimum(x, v0)))
            ni0.append(jnp.where(gt, rv, i0))
            nv0.append(jnp.maximum(v0, x))
        return (tuple(nv0), tuple(nv1), tuple(ni0))

    def argmax_row(buf, k, r, c):
        ms, i1s = c
        rv = jnp.full((16,), k * CHUNK + r, jnp.int32)
        nm, ni1 = [], []
        for g in range(NG):
            x = buf[r, pl.ds(g * 16, 16)]
            m, i1 = ms[g], i1s[g]
            gt = x > m
            ni1.append(jnp.where(gt, rv, i1))
            nm.append(jnp.maximum(m, x))
        return (tuple(nm), tuple(ni1))

    init1 = (tuple(neg for _ in range(NG)),
             tuple(neg for _ in range(NG)),
             tuple(zero_i for _ in range(NG)))
    v0s, v1s, i0s = _scan(in_hbm, col0, bufs, sems, init1, top2_row)

    init2 = (tuple(neg for _ in range(NG)),
             tuple(zero_i for _ in range(NG)))
    _, i1s = _scan(in_hbm, HALF + col0, bufs, sems, init2, argmax_row)

    pen_v = jnp.full((16,), -1e16, jnp.float32)
    zero_f = jnp.zeros((16,), jnp.float32)
    for g in range(NG):
        v0, v1 = v0s[g], v1s[g]
        pen = jnp.where(i0s[g] == i1s[g], pen_v, zero_f)
        out_v[pl.ds(g * 16, 16)] = jnp.maximum(v0 + v0 + pen, v0 + v1)
    pltpu.sync_copy(out_v, out_hbm.at[pl.ds(col0, CW)])


@jax.jit
def _run(inputs):
    mesh = plsc.VectorSubcoreMesh(core_axis_name="c", subcore_axis_name="s")
    f = pl.kernel(
        _body,
        out_type=jax.ShapeDtypeStruct((HALF,), jnp.float32),
        mesh=mesh,
        scratch_types=[
            pltpu.VMEM((CHUNK, CW), jnp.float32),
            pltpu.VMEM((CHUNK, CW), jnp.float32),
            pltpu.VMEM((CW,), jnp.float32),
            pltpu.SemaphoreType.DMA,
            pltpu.SemaphoreType.DMA,
        ],
    )
    return f(inputs)


def kernel(inputs):
    return _run(inputs).reshape(1, HALF)


# speedup vs baseline: 83.4845x; 1.0066x over previous
import functools

import jax
import jax.numpy as jnp
from jax import lax
from jax.experimental import pallas as pl
from jax.experimental.pallas import tpu as pltpu
from jax.experimental.pallas import tpu_sc as plsc

ROWS = 4096
COLS = 8192
HALF = COLS // 2
NW = 32
CW = HALF // NW
NG = CW // 16
CHUNK = 256
NCHUNK = ROWS // CHUNK


def _chunk_copy(in_hbm, col_base, k, buf, sem):
    return pltpu.make_async_copy(
        in_hbm.at[pl.ds(k * CHUNK, CHUNK), pl.ds(col_base, CW)], buf, sem)


def _scan(in_hbm, col_base, bufs, sems, init, row_body):
    _chunk_copy(in_hbm, col_base, 0, bufs[0], sems[0]).start()

    def outer(t, carry):
        for b in range(2):
            k = t * 2 + b
            _chunk_copy(in_hbm, col_base, k, bufs[b], sems[b]).wait()

            @pl.when(k + 1 < NCHUNK)
            def _():
                _chunk_copy(in_hbm, col_base, k + 1,
                            bufs[1 - b], sems[1 - b]).start()

            def row2(r, c):
                c = row_body(bufs[b], k, 2 * r, c)
                return row_body(bufs[b], k, 2 * r + 1, c)

            carry = lax.fori_loop(0, CHUNK // 2, row2, carry)
        return carry

    return lax.fori_loop(0, NCHUNK // 2, outer, init)


def _body(in_hbm, out_hbm, buf_a, buf_b, out_v, sem_a, sem_b):
    core = lax.axis_index("c")
    sub = lax.axis_index("s")
    wid = core * 16 + sub
    col0 = wid * CW

    bufs = (buf_a, buf_b)
    sems = (sem_a, sem_b)

    neg = jnp.full((16,), -jnp.inf, jnp.float32)
    zero_i = jnp.zeros((16,), jnp.int32)

    def top2_row(buf, k, r, c):
        v0s, v1s, i0s = c
        rv = jnp.full((16,), k * CHUNK + r, jnp.int32)
        nv0, nv1, ni0 = [], [], []
        for g in range(NG):
            x = buf[r, pl.ds(g * 16, 16)]
            v0, v1, i0 = v0s[g], v1s[g], i0s[g]
            gt = x > v0
            nv1.append(jnp.maximum(v1, jnp.minimum(x, v0)))
            ni0.append(jnp.where(gt, rv, i0))
            nv0.append(jnp.maximum(v0, x))
        return (tuple(nv0), tuple(nv1), tuple(ni0))

    def argmax_row(buf, k, r, c):
        ms, i1s = c
        rv = jnp.full((16,), k * CHUNK + r, jnp.int32)
        nm, ni1 = [], []
        for g in range(NG):
            x = buf[r, pl.ds(g * 16, 16)]
            m, i1 = ms[g], i1s[g]
            gt = x > m
            ni1.append(jnp.where(gt, rv, i1))
            nm.append(jnp.maximum(m, x))
        return (tuple(nm), tuple(ni1))

    init1 = (tuple(neg for _ in range(NG)),
             tuple(neg for _ in range(NG)),
             tuple(zero_i for _ in range(NG)))
    v0s, v1s, i0s = _scan(in_hbm, col0, bufs, sems, init1, top2_row)

    init2 = (tuple(neg for _ in range(NG)),
             tuple(zero_i for _ in range(NG)))
    _, i1s = _scan(in_hbm, HALF + col0, bufs, sems, init2, argmax_row)

    pen_v = jnp.full((16,), -1e16, jnp.float32)
    zero_f = jnp.zeros((16,), jnp.float32)
    for g in range(NG):
        v0, v1 = v0s[g], v1s[g]
        pen = jnp.where(i0s[g] == i1s[g], pen_v, zero_f)
        out_v[pl.ds(g * 16, 16)] = jnp.maximum(v0 + v0 + pen, v0 + v1)
    pltpu.sync_copy(out_v, out_hbm.at[pl.ds(col0, CW)])


@jax.jit
def _run(inputs):
    mesh = plsc.VectorSubcoreMesh(core_axis_name="c", subcore_axis_name="s")
    f = pl.kernel(
        _body,
        out_type=jax.ShapeDtypeStruct((HALF,), jnp.float32),
        mesh=mesh,
        scratch_types=[
            pltpu.VMEM((CHUNK, CW), jnp.float32),
            pltpu.VMEM((CHUNK, CW), jnp.float32),
            pltpu.VMEM((CW,), jnp.float32),
            pltpu.SemaphoreType.DMA,
            pltpu.SemaphoreType.DMA,
        ],
    )
    return f(inputs)


def kernel(inputs):
    return _run(inputs).reshape(1, HALF)


# speedup vs baseline: 114.2447x; 1.3685x over previous
import functools

import jax
import jax.numpy as jnp
from jax import lax
from jax.experimental import pallas as pl
from jax.experimental.pallas import tpu as pltpu
from jax.experimental.pallas import tpu_sc as plsc

ROWS = 4096
COLS = 8192
HALF = COLS // 2
NW = 32
CW = HALF // NW
NG = CW // 16
CHUNK = 256
NCHUNK = ROWS // CHUNK


def _chunk_copy(in_hbm, col_base, k, buf, sem):
    return pltpu.make_async_copy(
        in_hbm.at[pl.ds(k * CHUNK, CHUNK), pl.ds(col_base, CW)], buf, sem)


def _sc_body(in_hbm, v0_hbm, v1_hbm, i0_hbm, buf_a, buf_b,
             v0_v, v1_v, i0_v, sem_a, sem_b):
    core = lax.axis_index("c")
    sub = lax.axis_index("s")
    wid = core * 16 + sub
    col0 = wid * CW

    bufs = (buf_a, buf_b)
    sems = (sem_a, sem_b)

    neg = jnp.full((16,), -jnp.inf, jnp.float32)
    zero_i = jnp.zeros((16,), jnp.int32)

    def top2_row(buf, k, r, c):
        v0s, v1s, i0s = c
        rv = jnp.full((16,), k * CHUNK + r, jnp.int32)
        nv0, nv1, ni0 = [], [], []
        for g in range(NG):
            x = buf[r, pl.ds(g * 16, 16)]
            v0, v1, i0 = v0s[g], v1s[g], i0s[g]
            gt = x > v0
            nv1.append(jnp.maximum(v1, jnp.minimum(x, v0)))
            ni0.append(jnp.where(gt, rv, i0))
            nv0.append(jnp.maximum(v0, x))
        return (tuple(nv0), tuple(nv1), tuple(ni0))

    _chunk_copy(in_hbm, col0, 0, bufs[0], sems[0]).start()

    def outer(t, carry):
        for b in range(2):
            k = t * 2 + b
            _chunk_copy(in_hbm, col0, k, bufs[b], sems[b]).wait()

            @pl.when(k + 1 < NCHUNK)
            def _():
                _chunk_copy(in_hbm, col0, k + 1,
                            bufs[1 - b], sems[1 - b]).start()

            carry = lax.fori_loop(
                0, CHUNK, functools.partial(top2_row, bufs[b], k), carry)
        return carry

    init = (tuple(neg for _ in range(NG)),
            tuple(neg for _ in range(NG)),
            tuple(zero_i for _ in range(NG)))
    v0s, v1s, i0s = lax.fori_loop(0, NCHUNK // 2, outer, init)

    for g in range(NG):
        v0_v[pl.ds(g * 16, 16)] = v0s[g]
        v1_v[pl.ds(g * 16, 16)] = v1s[g]
        i0_v[pl.ds(g * 16, 16)] = i0s[g]
    pltpu.sync_copy(v0_v, v0_hbm.at[pl.ds(col0, CW)])
    pltpu.sync_copy(v1_v, v1_hbm.at[pl.ds(col0, CW)])
    pltpu.sync_copy(i0_v, i0_hbm.at[pl.ds(col0, CW)])


def _sc_top2(inputs):
    mesh = plsc.VectorSubcoreMesh(core_axis_name="c", subcore_axis_name="s")
    shp = jax.ShapeDtypeStruct((HALF,), jnp.float32)
    f = pl.kernel(
        _sc_body,
        out_type=(shp, shp, jax.ShapeDtypeStruct((HALF,), jnp.int32)),
        mesh=mesh,
        scratch_types=[
            pltpu.VMEM((CHUNK, CW), jnp.float32),
            pltpu.VMEM((CHUNK, CW), jnp.float32),
            pltpu.VMEM((CW,), jnp.float32),
            pltpu.VMEM((CW,), jnp.float32),
            pltpu.VMEM((CW,), jnp.int32),
            pltpu.SemaphoreType.DMA,
            pltpu.SemaphoreType.DMA,
        ],
    )
    return f(inputs)


TC_BLK = 512
BIG = 1 << 30


def _tc_argmax_body(x_ref, i1_ref):
    x = x_ref[...]
    m = jnp.max(x, axis=0)
    rows = lax.broadcasted_iota(jnp.int32, (ROWS, TC_BLK), 0)
    i1_ref[...] = jnp.min(jnp.where(x == m[None, :], rows, BIG),
                          axis=0, keepdims=True)


def _tc_argmax_half2(inputs):
    grid = HALF // TC_BLK
    return pl.pallas_call(
        _tc_argmax_body,
        grid=(grid,),
        in_specs=[pl.BlockSpec((ROWS, TC_BLK),
                               lambda j: (0, grid + j))],
        out_specs=pl.BlockSpec((1, TC_BLK), lambda j: (0, j)),
        out_shape=jax.ShapeDtypeStruct((1, HALF), jnp.int32),
    )(inputs)


def _tc_merge_body(v0_ref, v1_ref, i0_ref, i1_ref, out_ref):
    v0 = v0_ref[...]
    v1 = v1_ref[...]
    pen = jnp.where(i0_ref[...] == i1_ref[...],
                    jnp.float32(-1e16), jnp.float32(0.0))
    out_ref[...] = jnp.maximum(v0 + v0 + pen, v0 + v1)


def _tc_merge(v0, v1, i0, i1):
    return pl.pallas_call(
        _tc_merge_body,
        out_shape=jax.ShapeDtypeStruct((1, HALF), jnp.float32),
    )(v0.reshape(1, HALF), v1.reshape(1, HALF),
      i0.reshape(1, HALF), i1.reshape(1, HALF))


@jax.jit
def _run(inputs):
    i1 = _tc_argmax_half2(inputs)
    v0, v1, i0 = _sc_top2(inputs)
    return _tc_merge(v0, v1, i0, i1)


def kernel(inputs):
    return _run(inputs)


# speedup vs baseline: 114.2642x; 1.0002x over previous
import functools

import jax
import jax.numpy as jnp
from jax import lax
from jax.experimental import pallas as pl
from jax.experimental.pallas import tpu as pltpu
from jax.experimental.pallas import tpu_sc as plsc

ROWS = 4096
COLS = 8192
HALF = COLS // 2
SLABS_PER_CORE = 8
SC_COLS = 2 * SLABS_PER_CORE * 128
TC_COLS = HALF - SC_COLS
CW = 128
NG = CW // 16
HROWS = ROWS // 2
CHUNK = 256
NCHUNK = HROWS // CHUNK
TC_BLK = 512
SC_BLKS = SC_COLS // TC_BLK
BIG = 1 << 30


def _chunk_copy(in_hbm, row_base, col_base, k, buf, sem):
    return pltpu.make_async_copy(
        in_hbm.at[pl.ds(row_base + k * CHUNK, CHUNK), pl.ds(col_base, CW)],
        buf, sem)


def _sc_body(in_hbm, v0_hbm, v1_hbm, i0_hbm, buf_a, buf_b,
             v0_v, v1_v, i0_v, r_v0, r_v1, r_i0,
             sh_v0, sh_v1, sh_i0, sem_a, sem_b):
    core = lax.axis_index("c")
    sub = lax.axis_index("s")
    slab = sub % SLABS_PER_CORE
    upper = sub // SLABS_PER_CORE
    col0 = (core * SLABS_PER_CORE + slab) * CW
    row0 = upper * HROWS

    bufs = (buf_a, buf_b)
    sems = (sem_a, sem_b)

    neg = jnp.full((16,), -jnp.inf, jnp.float32)
    zero_i = jnp.zeros((16,), jnp.int32)

    def top2_row(buf, k, r, c):
        v0s, v1s, i0s = c
        rv = jnp.full((16,), 0, jnp.int32) + (row0 + k * CHUNK + r)
        nv0, nv1, ni0 = [], [], []
        for g in range(NG):
            x = buf[r, pl.ds(g * 16, 16)]
            v0, v1, i0 = v0s[g], v1s[g], i0s[g]
            gt = x > v0
            nv1.append(jnp.maximum(v1, jnp.minimum(x, v0)))
            ni0.append(jnp.where(gt, rv, i0))
            nv0.append(jnp.maximum(v0, x))
        return (tuple(nv0), tuple(nv1), tuple(ni0))

    _chunk_copy(in_hbm, row0, col0, 0, bufs[0], sems[0]).start()

    def outer(t, carry):
        for b in range(2):
            k = t * 2 + b
            _chunk_copy(in_hbm, row0, col0, k, bufs[b], sems[b]).wait()

            @pl.when(k + 1 < NCHUNK)
            def _():
                _chunk_copy(in_hbm, row0, col0, k + 1,
                            bufs[1 - b], sems[1 - b]).start()

            carry = lax.fori_loop(
                0, CHUNK, functools.partial(top2_row, bufs[b], k), carry)
        return carry

    init = (tuple(neg for _ in range(NG)),
            tuple(neg for _ in range(NG)),
            tuple(zero_i for _ in range(NG)))
    v0s, v1s, i0s = lax.fori_loop(0, NCHUNK // 2, outer, init)

    for g in range(NG):
        v0_v[pl.ds(g * 16, 16)] = v0s[g]
        v1_v[pl.ds(g * 16, 16)] = v1s[g]
        i0_v[pl.ds(g * 16, 16)] = i0s[g]

    @pl.when(upper == 1)
    def _():
        pltpu.sync_copy(v0_v, sh_v0.at[slab])
        pltpu.sync_copy(v1_v, sh_v1.at[slab])
        pltpu.sync_copy(i0_v, sh_i0.at[slab])

    plsc.subcore_barrier()

    @pl.when(upper == 0)
    def _():
        pltpu.sync_copy(sh_v0.at[slab], r_v0)
        pltpu.sync_copy(sh_v1.at[slab], r_v1)
        pltpu.sync_copy(sh_i0.at[slab], r_i0)
        for g in range(NG):
            a0, a1, ai = v0s[g], v1s[g], i0s[g]
            b0 = r_v0[pl.ds(g * 16, 16)]
            b1 = r_v1[pl.ds(g * 16, 16)]
            bi = r_i0[pl.ds(g * 16, 16)]
            gt = b0 > a0
            v0_v[pl.ds(g * 16, 16)] = jnp.maximum(a0, b0)
            v1_v[pl.ds(g * 16, 16)] = jnp.maximum(jnp.minimum(a0, b0),
                                                  jnp.maximum(a1, b1))
            i0_v[pl.ds(g * 16, 16)] = jnp.where(gt, bi, ai)
        pltpu.sync_copy(v0_v, v0_hbm.at[pl.ds(col0, CW)])
        pltpu.sync_copy(v1_v, v1_hbm.at[pl.ds(col0, CW)])
        pltpu.sync_copy(i0_v, i0_hbm.at[pl.ds(col0, CW)])


def _sc_top2(inputs):
    mesh = plsc.VectorSubcoreMesh(core_axis_name="c", subcore_axis_name="s")
    shp = jax.ShapeDtypeStruct((SC_COLS,), jnp.float32)
    f = pl.kernel(
        _sc_body,
        out_type=(shp, shp, jax.ShapeDtypeStruct((SC_COLS,), jnp.int32)),
        mesh=mesh,
        scratch_types=[
            pltpu.VMEM((CHUNK, CW), jnp.float32),
            pltpu.VMEM((CHUNK, CW), jnp.float32),
            pltpu.VMEM((CW,), jnp.float32),
            pltpu.VMEM((CW,), jnp.float32),
            pltpu.VMEM((CW,), jnp.int32),
            pltpu.VMEM((CW,), jnp.float32),
            pltpu.VMEM((CW,), jnp.float32),
            pltpu.VMEM((CW,), jnp.int32),
            pltpu.VMEM_SHARED((SLABS_PER_CORE, CW), jnp.float32),
            pltpu.VMEM_SHARED((SLABS_PER_CORE, CW), jnp.float32),
            pltpu.VMEM_SHARED((SLABS_PER_CORE, CW), jnp.int32),
            pltpu.SemaphoreType.DMA,
            pltpu.SemaphoreType.DMA,
        ],
    )
    return f(inputs)


def _colmax_argmax(x):
    m = jnp.max(x, axis=0)
    rows = lax.broadcasted_iota(jnp.int32, x.shape, 0)
    i = jnp.min(jnp.where(x == m[None, :], rows, BIG), axis=0)
    return m, i, rows


def _tc_full_body(x1_ref, x2_ref, out_ref):
    x1 = x1_ref[...]
    v0, i0, rows = _colmax_argmax(x1)
    v1 = jnp.max(jnp.where(rows == i0[None, :], -jnp.inf, x1), axis=0)
    x2 = x2_ref[...]
    _, i1, _ = _colmax_argmax(x2)
    pen = jnp.where(i0 == i1, jnp.float32(-1e16), jnp.float32(0.0))
    out_ref[...] = jnp.maximum(v0 + v0 + pen, v0 + v1)[None, :]


def _tc_full(inputs):
    grid = TC_COLS // TC_BLK
    return pl.pallas_call(
        _tc_full_body,
        grid=(grid,),
        in_specs=[
            pl.BlockSpec((ROWS, TC_BLK), lambda j: (0, SC_BLKS + j)),
            pl.BlockSpec((ROWS, TC_BLK),
                         lambda j: (0, HALF // TC_BLK + SC_BLKS + j)),
        ],
        out_specs=pl.BlockSpec((1, TC_BLK), lambda j: (0, j)),
        out_shape=jax.ShapeDtypeStruct((1, TC_COLS), jnp.float32),
    )(inputs, inputs)


def _tc_argmax_body(x_ref, i1_ref):
    x = x_ref[...]
    _, i, _ = _colmax_argmax(x)
    i1_ref[...] = i[None, :]


def _tc_argmax_sc_partners(inputs):
    return pl.pallas_call(
        _tc_argmax_body,
        grid=(SC_BLKS,),
        in_specs=[pl.BlockSpec((ROWS, TC_BLK),
                               lambda j: (0, HALF // TC_BLK + j))],
        out_specs=pl.BlockSpec((1, TC_BLK), lambda j: (0, j)),
        out_shape=jax.ShapeDtypeStruct((1, SC_COLS), jnp.int32),
    )(inputs)


def _tc_merge_body(v0_ref, v1_ref, i0_ref, i1_ref, tc_ref, out_ref):
    v0 = v0_ref[...]
    v1 = v1_ref[...]
    pen = jnp.where(i0_ref[...] == i1_ref[...],
                    jnp.float32(-1e16), jnp.float32(0.0))
    out_ref[:, :SC_COLS] = jnp.maximum(v0 + v0 + pen, v0 + v1)
    out_ref[:, SC_COLS:] = tc_ref[...]


def _tc_merge(v0, v1, i0, i1, tc_out):
    return pl.pallas_call(
        _tc_merge_body,
        out_shape=jax.ShapeDtypeStruct((1, HALF), jnp.float32),
    )(v0.reshape(1, SC_COLS), v1.reshape(1, SC_COLS),
      i0.reshape(1, SC_COLS), i1, tc_out)


@jax.jit
def _run(inputs):
    tc_out = _tc_full(inputs)
    i1 = _tc_argmax_sc_partners(inputs)
    v0, v1, i0 = _sc_top2(inputs)
    return _tc_merge(v0, v1, i0, i1, tc_out)


def kernel(inputs):
    return _run(inputs)
